# baseline (speedup 1.0000x reference)
import functools

import jax
import jax.numpy as jnp
from jax import lax
from jax.experimental import pallas as pl
from jax.experimental.pallas import tpu as pltpu
from jax.experimental.pallas import tpu_sc as plsc

EMBED_DIM = 32
BATCH = 16384
NUM_CORES = 2
NUM_SUBCORES = 16
NUM_WORKERS = NUM_CORES * NUM_SUBCORES
B_PER_W = BATCH // NUM_WORKERS
CHUNK = 128
N_CHUNKS = B_PER_W // CHUNK


def _gather_body(uidx_hbm, midx_hbm, utab_hbm, mtab_hbm, uout_hbm, mout_hbm,
                 uidx_v, midx_v, urows_v, mrows_v, sem_u, sem_m):
    wid = lax.axis_index("s") * NUM_CORES + lax.axis_index("c")
    base = wid * B_PER_W
    pltpu.sync_copy(uidx_hbm.at[pl.ds(base, B_PER_W)], uidx_v)
    pltpu.sync_copy(midx_hbm.at[pl.ds(base, B_PER_W)], midx_v)
    for j in range(N_CHUNKS):
        s = j * CHUNK
        pltpu.async_copy(utab_hbm.at[uidx_v.at[pl.ds(s, CHUNK)]],
                         urows_v.at[pl.ds(s, CHUNK)], sem_u)
        pltpu.async_copy(mtab_hbm.at[midx_v.at[pl.ds(s, CHUNK)]],
                         mrows_v.at[pl.ds(s, CHUNK)], sem_m)
    for j in range(N_CHUNKS):
        s = j * CHUNK
        pltpu.make_async_copy(utab_hbm.at[uidx_v.at[pl.ds(s, CHUNK)]],
                              urows_v.at[pl.ds(s, CHUNK)], sem_u).wait()
        pltpu.make_async_copy(mtab_hbm.at[midx_v.at[pl.ds(s, CHUNK)]],
                              mrows_v.at[pl.ds(s, CHUNK)], sem_m).wait()
    pltpu.sync_copy(urows_v, uout_hbm.at[pl.ds(base, B_PER_W)])
    pltpu.sync_copy(mrows_v, mout_hbm.at[pl.ds(base, B_PER_W)])


@jax.jit
def _sc_gather(user_ids, movie_ids, user_table, movie_table):
    mesh = plsc.VectorSubcoreMesh(core_axis_name="c", subcore_axis_name="s")
    fn = functools.partial(
        pl.kernel,
        mesh=mesh,
        compiler_params=pltpu.CompilerParams(use_tc_tiling_on_sc=False),
        out_type=[
            jax.ShapeDtypeStruct((BATCH, EMBED_DIM), jnp.float32),
            jax.ShapeDtypeStruct((BATCH, EMBED_DIM), jnp.float32),
        ],
        scratch_types=[
            pltpu.VMEM((B_PER_W,), jnp.int32),
            pltpu.VMEM((B_PER_W,), jnp.int32),
            pltpu.VMEM((B_PER_W, EMBED_DIM), jnp.float32),
            pltpu.VMEM((B_PER_W, EMBED_DIM), jnp.float32),
            pltpu.SemaphoreType.DMA,
            pltpu.SemaphoreType.DMA,
        ],
    )(_gather_body)
    return fn(user_ids, movie_ids, user_table, movie_table)


def _mlp_body(u_ref, m_ref, w1a_ref, w1b_ref, b1_ref, w2_ref, b2_ref,
              w3_ref, b3_ref, out_ref):
    h = jnp.dot(u_ref[...], w1a_ref[...], preferred_element_type=jnp.float32)
    h = h + jnp.dot(m_ref[...], w1b_ref[...], preferred_element_type=jnp.float32)
    h = jnp.maximum(h + b1_ref[...], 0.0)
    h2 = jnp.dot(h, w2_ref[...], preferred_element_type=jnp.float32)
    h2 = jnp.maximum(h2 + b2_ref[...], 0.0)
    out_ref[...] = jnp.sum(h2 * w3_ref[...], axis=1) + b3_ref[0, 0]


def _tc_mlp(u, m, W1, b1, W2, b2, W3, b3, bm=2048):
    w1a = W1[:EMBED_DIM]
    w1b = W1[EMBED_DIM:]
    b1r = b1.reshape(1, -1)
    b2r = b2.reshape(1, -1)
    w3r = W3.reshape(1, -1)
    b3r = b3.reshape(1, 1)
    grid = (BATCH // bm,)
    return pl.pallas_call(
        _mlp_body,
        grid=grid,
        in_specs=[
            pl.BlockSpec((bm, EMBED_DIM), lambda i: (i, 0)),
            pl.BlockSpec((bm, EMBED_DIM), lambda i: (i, 0)),
            pl.BlockSpec(w1a.shape, lambda i: (0, 0)),
            pl.BlockSpec(w1b.shape, lambda i: (0, 0)),
            pl.BlockSpec(b1r.shape, lambda i: (0, 0)),
            pl.BlockSpec(W2.shape, lambda i: (0, 0)),
            pl.BlockSpec(b2r.shape, lambda i: (0, 0)),
            pl.BlockSpec(w3r.shape, lambda i: (0, 0)),
            pl.BlockSpec(b3r.shape, lambda i: (0, 0)),
        ],
        out_specs=pl.BlockSpec((bm,), lambda i: (i,)),
        out_shape=jax.ShapeDtypeStruct((BATCH,), jnp.float32),
    )(u, m, w1a, w1b, b1r, W2, b2r, w3r, b3r)


def kernel(user_ids, movie_ids, user_table, movie_table, W1, b1, W2, b2, W3, b3):
    u, m = _sc_gather(user_ids.astype(jnp.int32), movie_ids.astype(jnp.int32),
                      user_table, movie_table)
    return _tc_mlp(u, m, W1, b1, W2, b2, W3, b3)
